# baseline (speedup 1.0000x reference)
import functools

import jax
import jax.numpy as jnp
from jax import lax
from jax.experimental import pallas as pl
from jax.experimental.pallas import tpu as pltpu
from jax.experimental.pallas import tpu_sc as plsc

B = 16384
F = 26
CARD = 1000
D = 128

NC = 2
NS = 16
NW = NC * NS
LANES = 16

TOTAL_ROWS = B * F
ROWS_W = TOTAL_ROWS // NW
CHUNK = 64
NBUF = 2
HALF_F = F // 2
PHASES = 2
XV_W = ROWS_W // PHASES
CPP = XV_W // CHUNK
BREP = CHUNK
IDXB_W = (CARD // LANES + 1) * LANES


def _sc_gather_body(
    x_hbm, tbl_hbm, b_hbm, out_hbm, xv, idxb, shared, bufs, gsems, wsems, tsem
):
    cid = lax.axis_index("c")
    sid = lax.axis_index("s")
    wid = cid * NS + sid
    base = wid * ROWS_W

    def _stage_idx(pbase):
        pltpu.sync_copy(x_hbm.at[pl.ds(pbase, XV_W)], xv)

        @pl.loop(0, CPP)
        def _idx_loop(j):
            f = lax.div(pbase + j * CHUNK, B)
            off = (f - cid * HALF_F) * CARD
            for s in range(CHUNK // LANES):
                sl = pl.ds(j * CHUNK + s * LANES, LANES)
                xv[sl] = xv[sl] + off

    @pl.when(sid < HALF_F)
    def _():
        pltpu.async_copy(
            tbl_hbm.at[pl.ds((cid * HALF_F + sid) * CARD, CARD)],
            shared.at[pl.ds(sid * CARD, CARD)],
            tsem,
        )

        pltpu.sync_copy(
            b_hbm.at[pl.ds(cid * HALF_F + sid, 1)], bufs[0].at[pl.ds(0, 1)]
        )
        brow = [bufs[0][0, pl.ds(k * LANES, LANES)] for k in range(D // LANES)]
        for r in range(1, BREP):
            for k in range(D // LANES):
                bufs[0][r, pl.ds(k * LANES, LANES)] = brow[k]

        iota = lax.iota(jnp.int32, LANES)

        @pl.loop(0, CARD // LANES + 1)
        def _idxb_loop(i):
            idxb[pl.ds(i * LANES, LANES)] = sid * CARD + i * LANES + iota

        _stage_idx(base)

        pltpu.make_async_copy(
            tbl_hbm.at[pl.ds((cid * HALF_F + sid) * CARD, CARD)],
            shared.at[pl.ds(sid * CARD, CARD)],
            tsem,
        ).wait()

        for k in range(CARD // BREP):
            pltpu.async_copy(
                bufs[0],
                shared.at[idxb.at[pl.ds(k * BREP, BREP)]],
                wsems[0],
                add=True,
            )
        pltpu.async_copy(
            bufs[0].at[pl.ds(0, CARD % BREP)],
            shared.at[idxb.at[pl.ds(CARD - CARD % BREP, CARD % BREP)]],
            wsems[0],
            add=True,
        )
        for k in range(CARD // BREP):
            pltpu.make_async_copy(
                bufs[0], shared.at[idxb.at[pl.ds(k * BREP, BREP)]], wsems[0]
            ).wait()
        pltpu.make_async_copy(
            bufs[0].at[pl.ds(0, CARD % BREP)],
            shared.at[idxb.at[pl.ds(CARD - CARD % BREP, CARD % BREP)]],
            wsems[0],
        ).wait()

    @pl.when(sid >= HALF_F)
    def _():
        _stage_idx(base)

    plsc.subcore_barrier()

    def _start_gather(g, slot):
        pltpu.async_copy(
            shared.at[xv.at[pl.ds(g * CHUNK, CHUNK)]], bufs[slot], gsems[slot]
        )

    def _wait_gather(g, slot):
        pltpu.make_async_copy(
            shared.at[xv.at[pl.ds(g * CHUNK, CHUNK)]], bufs[slot], gsems[slot]
        ).wait()

    for phase in range(PHASES):
        pbase = base + phase * XV_W

        def _start_write(g, slot, pbase=pbase):
            pltpu.async_copy(
                bufs[slot],
                out_hbm.at[pl.ds(pbase + g * CHUNK, CHUNK)],
                wsems[slot],
            )

        def _wait_write(g, slot, pbase=pbase):
            pltpu.make_async_copy(
                bufs[slot],
                out_hbm.at[pl.ds(pbase + g * CHUNK, CHUNK)],
                wsems[slot],
            ).wait()

        if phase > 0:
            _stage_idx(pbase)

        _start_gather(0, 0)

        @pl.loop(0, CPP, step=NBUF, unroll=2)
        def _gather_loop(g0):
            for slot in range(NBUF):
                g = g0 + slot
                nxt = (slot + 1) % NBUF

                @pl.when(g + 1 < CPP)
                def _():
                    @pl.when(g >= NBUF - 1)
                    def _():
                        _wait_write(g + 1 - NBUF, nxt)

                    _start_gather(g + 1, nxt)

                _wait_gather(g, slot)
                _start_write(g, slot)

        for slot in range(NBUF):
            _wait_write(CPP - NBUF + slot, slot)


@functools.partial(
    pl.kernel,
    out_type=jax.ShapeDtypeStruct((TOTAL_ROWS, D), jnp.float32),
    mesh=plsc.VectorSubcoreMesh(core_axis_name="c", subcore_axis_name="s"),
    scratch_types=[
        pltpu.VMEM((XV_W,), jnp.int32),
        pltpu.VMEM((IDXB_W,), jnp.int32),
        pltpu.VMEM_SHARED((HALF_F * CARD, D), jnp.float32),
        [pltpu.VMEM((CHUNK, D), jnp.float32) for _ in range(NBUF)],
        [pltpu.SemaphoreType.DMA for _ in range(NBUF)],
        [pltpu.SemaphoreType.DMA for _ in range(NBUF)],
        pltpu.SemaphoreType.DMA,
    ],
)
def _sc_gather(x_hbm, tbl_hbm, b_hbm, out_hbm, xv, idxb, shared, bufs, gsems, wsems, tsem):
    _sc_gather_body(
        x_hbm, tbl_hbm, b_hbm, out_hbm, xv, idxb, shared, bufs, gsems, wsems, tsem
    )


@jax.jit
def kernel(x, tables, b):
    tbl_flat = tables.reshape(F * CARD, D)
    x_fm = jnp.swapaxes(x, 0, 1).reshape(-1).astype(jnp.int32)
    out = _sc_gather(x_fm, tbl_flat, b)
    return jnp.swapaxes(out.reshape(F, B, D), 0, 1)


# speedup vs baseline: 1.0024x; 1.0024x over previous
import functools

import jax
import jax.numpy as jnp
from jax import lax
from jax.experimental import pallas as pl
from jax.experimental.pallas import tpu as pltpu
from jax.experimental.pallas import tpu_sc as plsc

B = 16384
F = 26
CARD = 1000
D = 128

NC = 2
NS = 16
NW = NC * NS
LANES = 16

TOTAL_ROWS = B * F
ROWS_W = TOTAL_ROWS // NW
CHUNK = 64
NBUF = 2
HALF_F = F // 2
PHASES = 2
XV_W = ROWS_W // PHASES
CPP = XV_W // CHUNK
BREP = CHUNK
IDXB_W = (CARD // LANES + 1) * LANES


def _sc_gather_body(
    x_hbm, tbl_hbm, b_hbm, out_hbm, xv, idxb, shared, bufs, gsems, wsems, tsem
):
    cid = lax.axis_index("c")
    sid = lax.axis_index("s")
    wid = cid * NS + sid
    base = wid * ROWS_W

    def _stage_idx(pbase):
        pltpu.sync_copy(x_hbm.at[pl.ds(pbase, XV_W)], xv)

        @pl.loop(0, CPP)
        def _idx_loop(j):
            f = lax.div(pbase + j * CHUNK, B)
            off = (f - cid * HALF_F) * CARD
            for s in range(CHUNK // LANES):
                sl = pl.ds(j * CHUNK + s * LANES, LANES)
                xv[sl] = xv[sl] + off

    @pl.when(sid < HALF_F)
    def _():
        pltpu.async_copy(
            tbl_hbm.at[pl.ds((cid * HALF_F + sid) * CARD, CARD)],
            shared.at[pl.ds(sid * CARD, CARD)],
            tsem,
        )

        pltpu.sync_copy(
            b_hbm.at[pl.ds(cid * HALF_F + sid, 1)], bufs[0].at[pl.ds(0, 1)]
        )
        brow = [bufs[0][0, pl.ds(k * LANES, LANES)] for k in range(D // LANES)]
        for r in range(1, BREP):
            for k in range(D // LANES):
                bufs[0][r, pl.ds(k * LANES, LANES)] = brow[k]

        iota = lax.iota(jnp.int32, LANES)

        @pl.loop(0, CARD // LANES + 1)
        def _idxb_loop(i):
            idxb[pl.ds(i * LANES, LANES)] = sid * CARD + i * LANES + iota

        _stage_idx(base)

        pltpu.make_async_copy(
            tbl_hbm.at[pl.ds((cid * HALF_F + sid) * CARD, CARD)],
            shared.at[pl.ds(sid * CARD, CARD)],
            tsem,
        ).wait()

        for k in range(CARD // BREP):
            pltpu.async_copy(
                bufs[0],
                shared.at[idxb.at[pl.ds(k * BREP, BREP)]],
                wsems[0],
                add=True,
            )
        pltpu.async_copy(
            bufs[0].at[pl.ds(0, CARD % BREP)],
            shared.at[idxb.at[pl.ds(CARD - CARD % BREP, CARD % BREP)]],
            wsems[0],
            add=True,
        )
        for k in range(CARD // BREP):
            pltpu.make_async_copy(
                bufs[0], shared.at[idxb.at[pl.ds(k * BREP, BREP)]], wsems[0]
            ).wait()
        pltpu.make_async_copy(
            bufs[0].at[pl.ds(0, CARD % BREP)],
            shared.at[idxb.at[pl.ds(CARD - CARD % BREP, CARD % BREP)]],
            wsems[0],
        ).wait()

    @pl.when(sid >= HALF_F)
    def _():
        _stage_idx(base)

    plsc.subcore_barrier()

    def _start_gather(g, slot):
        pltpu.async_copy(
            shared.at[xv.at[pl.ds(g * CHUNK, CHUNK)]], bufs[slot], gsems[slot]
        )

    def _wait_gather(g, slot):
        pltpu.make_async_copy(
            shared.at[xv.at[pl.ds(g * CHUNK, CHUNK)]], bufs[slot], gsems[slot]
        ).wait()

    for phase in range(PHASES):
        pbase = base + phase * XV_W

        def _start_write(g, slot, pbase=pbase):
            pltpu.async_copy(
                bufs[slot],
                out_hbm.at[pl.ds(pbase + g * CHUNK, CHUNK)],
                wsems[slot],
            )

        def _wait_write(g, slot, pbase=pbase):
            pltpu.make_async_copy(
                bufs[slot],
                out_hbm.at[pl.ds(pbase + g * CHUNK, CHUNK)],
                wsems[slot],
            ).wait()

        if phase > 0:
            _stage_idx(pbase)

        _start_gather(0, 0)

        @pl.loop(0, CPP, step=NBUF)
        def _gather_loop(g0):
            for slot in range(NBUF):
                g = g0 + slot
                nxt = (slot + 1) % NBUF

                @pl.when(g + 1 < CPP)
                def _():
                    @pl.when(g >= NBUF - 1)
                    def _():
                        _wait_write(g + 1 - NBUF, nxt)

                    _start_gather(g + 1, nxt)

                _wait_gather(g, slot)
                _start_write(g, slot)

        for slot in range(NBUF):
            _wait_write(CPP - NBUF + slot, slot)


@functools.partial(
    pl.kernel,
    out_type=jax.ShapeDtypeStruct((TOTAL_ROWS, D), jnp.float32),
    mesh=plsc.VectorSubcoreMesh(core_axis_name="c", subcore_axis_name="s"),
    scratch_types=[
        pltpu.VMEM((XV_W,), jnp.int32),
        pltpu.VMEM((IDXB_W,), jnp.int32),
        pltpu.VMEM_SHARED((HALF_F * CARD, D), jnp.float32),
        [pltpu.VMEM((CHUNK, D), jnp.float32) for _ in range(NBUF)],
        [pltpu.SemaphoreType.DMA for _ in range(NBUF)],
        [pltpu.SemaphoreType.DMA for _ in range(NBUF)],
        pltpu.SemaphoreType.DMA,
    ],
)
def _sc_gather(x_hbm, tbl_hbm, b_hbm, out_hbm, xv, idxb, shared, bufs, gsems, wsems, tsem):
    _sc_gather_body(
        x_hbm, tbl_hbm, b_hbm, out_hbm, xv, idxb, shared, bufs, gsems, wsems, tsem
    )


@jax.jit
def kernel(x, tables, b):
    tbl_flat = tables.reshape(F * CARD, D)
    x_fm = jnp.swapaxes(x, 0, 1).reshape(-1).astype(jnp.int32)
    out = _sc_gather(x_fm, tbl_flat, b)
    return jnp.swapaxes(out.reshape(F, B, D), 0, 1)
